# baseline (speedup 1.0000x reference)
import functools

import jax
import jax.numpy as jnp
from jax import lax
from jax.experimental import pallas as pl
from jax.experimental.pallas import tpu as pltpu
from jax.experimental.pallas import tpu_sc as plsc

VOCAB = 1000000
EMB = 64
B = 16384
T = 200

NC = 2
NS = 16
NW = NC * NS
SEQ_PER_W = B // NW
IDS_CHUNK = 64
N_CHUNKS = SEQ_PER_W // IDS_CHUNK
G0 = 96
G1 = T - G0


def _seq_gather(table_hbm, ids_v, rows_v, sem, j):
    d0 = pltpu.async_copy(table_hbm.at[ids_v.at[j, pl.ds(0, G0)]],
                          rows_v.at[pl.ds(0, G0)], sem)
    d1 = pltpu.async_copy(table_hbm.at[ids_v.at[j, pl.ds(G0, G1)]],
                          rows_v.at[pl.ds(G0, G1)], sem)
    return d0, d1


def _seq_wait(table_hbm, ids_v, rows_v, sem, j):
    pltpu.make_async_copy(table_hbm.at[ids_v.at[j, pl.ds(0, G0)]],
                          rows_v.at[pl.ds(0, G0)], sem).wait()
    pltpu.make_async_copy(table_hbm.at[ids_v.at[j, pl.ds(G0, G1)]],
                          rows_v.at[pl.ds(G0, G1)], sem).wait()


NBUF = 4


@functools.partial(
    pl.kernel,
    out_type=jax.ShapeDtypeStruct((B, EMB), jnp.float32),
    mesh=plsc.VectorSubcoreMesh(core_axis_name="c", subcore_axis_name="s"),
    scratch_types=[
        pltpu.VMEM((IDS_CHUNK, T), jnp.int32),
        [pltpu.VMEM((T, EMB), jnp.float32) for _ in range(NBUF)],
        pltpu.VMEM((SEQ_PER_W, EMB), jnp.float32),
        [pltpu.SemaphoreType.DMA for _ in range(NBUF)],
    ],
    compiler_params=pltpu.CompilerParams(use_tc_tiling_on_sc=False),
)
def _pool(ids_hbm, table_hbm, out_hbm, ids_v, rows, out_v, sems):
    wid = lax.axis_index("s") * NC + lax.axis_index("c")
    seq0 = wid * SEQ_PER_W

    def accum(rows_p, s):
        def body(t, acc):
            return tuple(acc[j] + rows_p[t, pl.ds(16 * j, 16)]
                         for j in range(4))
        acc = lax.fori_loop(
            0, T, body,
            tuple(jnp.zeros((16,), jnp.float32) for _ in range(4)),
            unroll=10)
        for j in range(4):
            out_v[s, pl.ds(16 * j, 16)] = acc[j]

    def chunk_body(c, carry):
        del carry
        pltpu.sync_copy(
            ids_hbm.at[pl.ds(pl.multiple_of(seq0 + c * IDS_CHUNK, 8),
                             IDS_CHUNK), :],
            ids_v)
        seq_base = c * IDS_CHUNK

        for j in range(NBUF - 1):
            _seq_gather(table_hbm, ids_v, rows[j], sems[j], j)

        def step(i, carry):
            del carry
            for p in range(NBUF):
                j = i * NBUF + p
                _seq_wait(table_hbm, ids_v, rows[p], sems[p], j)

                @pl.when(j + NBUF - 1 < IDS_CHUNK)
                def _():
                    _seq_gather(table_hbm, ids_v, rows[(p + NBUF - 1) % NBUF],
                                sems[(p + NBUF - 1) % NBUF], j + NBUF - 1)
                accum(rows[p], seq_base + j)
            return 0

        lax.fori_loop(0, IDS_CHUNK // NBUF, step, 0)
        return 0

    lax.fori_loop(0, N_CHUNKS, chunk_body, 0)
    pltpu.sync_copy(out_v,
                    out_hbm.at[pl.ds(pl.multiple_of(wid * SEQ_PER_W, 8),
                                     SEQ_PER_W)])


def _mlp_body(x_ref, w1_ref, b1_ref, w2_ref, b2_ref, o_ref):
    x = x_ref[...]
    h = lax.dot_general(x, w1_ref[...], (((1,), (1,)), ((), ())),
                        preferred_element_type=jnp.float32)
    h = jnp.maximum(h * (1.0 / T) + b1_ref[...], 0.0)
    o_ref[...] = lax.dot_general(h, w2_ref[...], (((1,), (1,)), ((), ())),
                                 preferred_element_type=jnp.float32) + b2_ref[...]


_BLK = 2048


def _mlp(sums, W1, b1, W2, b2):
    grid = B // _BLK
    return pl.pallas_call(
        _mlp_body,
        grid=(grid,),
        in_specs=[
            pl.BlockSpec((_BLK, EMB), lambda i: (i, 0)),
            pl.BlockSpec((EMB, EMB), lambda i: (0, 0)),
            pl.BlockSpec((1, EMB), lambda i: (0, 0)),
            pl.BlockSpec((EMB, EMB), lambda i: (0, 0)),
            pl.BlockSpec((1, EMB), lambda i: (0, 0)),
        ],
        out_specs=pl.BlockSpec((_BLK, EMB), lambda i: (i, 0)),
        out_shape=jax.ShapeDtypeStruct((B, EMB), jnp.float32),
    )(sums, W1, b1, W2, b2)


def kernel(token_ids, table, W1, b1, W2, b2):
    sums = _pool(token_ids, table)
    return _mlp(sums, W1, b1.reshape(1, EMB), W2, b2.reshape(1, EMB))


# speedup vs baseline: 1.0601x; 1.0601x over previous
import functools

import jax
import jax.numpy as jnp
from jax import lax
from jax.experimental import pallas as pl
from jax.experimental.pallas import tpu as pltpu
from jax.experimental.pallas import tpu_sc as plsc

VOCAB = 1000000
EMB = 64
B = 16384
T = 200

NC = 2
NS = 16
NW = NC * NS
SEQ_PER_W = B // NW
IDS_CHUNK = 64
N_CHUNKS = SEQ_PER_W // IDS_CHUNK
G0 = 96
G1 = T - G0


def _seq_gather(table_hbm, ids_v, rows_v, sem, j):
    d0 = pltpu.async_copy(table_hbm.at[ids_v.at[j, pl.ds(0, G0)]],
                          rows_v.at[pl.ds(0, G0)], sem)
    d1 = pltpu.async_copy(table_hbm.at[ids_v.at[j, pl.ds(G0, G1)]],
                          rows_v.at[pl.ds(G0, G1)], sem)
    return d0, d1


def _seq_wait(table_hbm, ids_v, rows_v, sem, j):
    pltpu.make_async_copy(table_hbm.at[ids_v.at[j, pl.ds(0, G0)]],
                          rows_v.at[pl.ds(0, G0)], sem).wait()
    pltpu.make_async_copy(table_hbm.at[ids_v.at[j, pl.ds(G0, G1)]],
                          rows_v.at[pl.ds(G0, G1)], sem).wait()


NBUF = 4


@functools.partial(
    pl.kernel,
    out_type=jax.ShapeDtypeStruct((B, EMB), jnp.float32),
    mesh=plsc.VectorSubcoreMesh(core_axis_name="c", subcore_axis_name="s"),
    scratch_types=[
        pltpu.VMEM((IDS_CHUNK, T), jnp.int32),
        [pltpu.VMEM((T, EMB), jnp.float32) for _ in range(NBUF)],
        pltpu.VMEM((SEQ_PER_W, EMB), jnp.float32),
        [pltpu.SemaphoreType.DMA for _ in range(NBUF)],
    ],
    compiler_params=pltpu.CompilerParams(use_tc_tiling_on_sc=False),
)
def _pool(ids_hbm, table_hbm, out_hbm, ids_v, rows, out_v, sems):
    wid = lax.axis_index("s") * NC + lax.axis_index("c")
    seq0 = wid * SEQ_PER_W

    def accum(rows_p, s):
        def body(t, acc):
            return tuple(acc[j] + rows_p[t, pl.ds(16 * j, 16)]
                         for j in range(4))
        acc = lax.fori_loop(
            0, T, body,
            tuple(jnp.zeros((16,), jnp.float32) for _ in range(4)),
            unroll=10)
        for j in range(4):
            out_v[s, pl.ds(16 * j, 16)] = acc[j]

    def chunk_body(c, carry):
        del carry
        pltpu.sync_copy(
            ids_hbm.at[pl.ds(pl.multiple_of(seq0 + c * IDS_CHUNK, 8),
                             IDS_CHUNK), :],
            ids_v)
        seq_base = c * IDS_CHUNK

        for j in range(NBUF - 1):
            _seq_gather(table_hbm, ids_v, rows[j], sems[j], j)

        def step(i, carry):
            del carry
            for p in range(NBUF):
                j = i * NBUF + p
                _seq_wait(table_hbm, ids_v, rows[p], sems[p], j)

                @pl.when(j + NBUF - 1 < IDS_CHUNK)
                def _():
                    _seq_gather(table_hbm, ids_v, rows[(p + NBUF - 1) % NBUF],
                                sems[(p + NBUF - 1) % NBUF], j + NBUF - 1)
                accum(rows[p], seq_base + j)
            return 0

        lax.fori_loop(0, IDS_CHUNK // NBUF, step, 0)
        return 0

    lax.fori_loop(0, N_CHUNKS, chunk_body, 0)
    pltpu.sync_copy(out_v,
                    out_hbm.at[pl.ds(pl.multiple_of(wid * SEQ_PER_W, 8),
                                     SEQ_PER_W)])


_TRB = 2048


def _transpose_body(x_ref, o_ref):
    x3 = x_ref[...].T.reshape(_TRB // 2, 2, EMB)
    o_ref[...] = jnp.concatenate([x3[:, 0, :], x3[:, 1, :]], axis=1)


def _linearize(tT):
    nblk = (VOCAB + _TRB - 1) // _TRB
    return pl.pallas_call(
        _transpose_body,
        grid=(nblk,),
        in_specs=[pl.BlockSpec((EMB, _TRB), lambda i: (0, i))],
        out_specs=pl.BlockSpec((_TRB // 2, 128), lambda i: (i, 0)),
        out_shape=jax.ShapeDtypeStruct((VOCAB // 2, 128), jnp.float32),
    )(tT)


def _mlp_body(x_ref, w1_ref, b1_ref, w2_ref, b2_ref, o_ref):
    x = x_ref[...]
    h = lax.dot_general(x, w1_ref[...], (((1,), (1,)), ((), ())),
                        preferred_element_type=jnp.float32)
    h = jnp.maximum(h * (1.0 / T) + b1_ref[...], 0.0)
    o_ref[...] = lax.dot_general(h, w2_ref[...], (((1,), (1,)), ((), ())),
                                 preferred_element_type=jnp.float32) + b2_ref[...]


_BLK = 2048


def _mlp(sums, W1, b1, W2, b2):
    grid = B // _BLK
    return pl.pallas_call(
        _mlp_body,
        grid=(grid,),
        in_specs=[
            pl.BlockSpec((_BLK, EMB), lambda i: (i, 0)),
            pl.BlockSpec((EMB, EMB), lambda i: (0, 0)),
            pl.BlockSpec((1, EMB), lambda i: (0, 0)),
            pl.BlockSpec((EMB, EMB), lambda i: (0, 0)),
            pl.BlockSpec((1, EMB), lambda i: (0, 0)),
        ],
        out_specs=pl.BlockSpec((_BLK, EMB), lambda i: (i, 0)),
        out_shape=jax.ShapeDtypeStruct((B, EMB), jnp.float32),
    )(sums, W1, b1, W2, b2)


def kernel(token_ids, table, W1, b1, W2, b2):
    t_lin = _linearize(table.T).reshape(VOCAB, EMB)
    sums = _pool(token_ids, t_lin)
    return _mlp(sums, W1, b1.reshape(1, EMB), W2, b2.reshape(1, EMB))


# speedup vs baseline: 1.1512x; 1.0859x over previous
import functools

import jax
import jax.numpy as jnp
from jax import lax
from jax.experimental import pallas as pl
from jax.experimental.pallas import tpu as pltpu
from jax.experimental.pallas import tpu_sc as plsc

VOCAB = 1000000
EMB = 64
B = 16384
T = 200

NC = 2
NS = 16
NW = NC * NS
SEQ_PER_W = B // NW
IDS_CHUNK = 64
N_CHUNKS = SEQ_PER_W // IDS_CHUNK
G0 = 96
G1 = T - G0


def _seq_gather(table_hbm, ids_v, rows_v, sem, j):
    off = pl.multiple_of(j * T, 8)
    d0 = pltpu.async_copy(table_hbm.at[ids_v.at[pl.ds(off, G0)]],
                          rows_v.at[pl.ds(0, G0)], sem)
    d1 = pltpu.async_copy(table_hbm.at[ids_v.at[pl.ds(off + G0, G1)]],
                          rows_v.at[pl.ds(G0, G1)], sem)
    return d0, d1


def _seq_wait(table_hbm, ids_v, rows_v, sem, j):
    off = pl.multiple_of(j * T, 8)
    pltpu.make_async_copy(table_hbm.at[ids_v.at[pl.ds(off, G0)]],
                          rows_v.at[pl.ds(0, G0)], sem).wait()
    pltpu.make_async_copy(table_hbm.at[ids_v.at[pl.ds(off + G0, G1)]],
                          rows_v.at[pl.ds(G0, G1)], sem).wait()


NBUF = 4


@functools.partial(
    pl.kernel,
    out_type=jax.ShapeDtypeStruct((B, EMB), jnp.float32),
    mesh=plsc.VectorSubcoreMesh(core_axis_name="c", subcore_axis_name="s"),
    scratch_types=[
        pltpu.VMEM((IDS_CHUNK * T,), jnp.int32),
        [pltpu.VMEM((T, EMB), jnp.float32) for _ in range(NBUF)],
        pltpu.VMEM((SEQ_PER_W, EMB), jnp.float32),
        [pltpu.SemaphoreType.DMA for _ in range(NBUF)],
    ],
    compiler_params=pltpu.CompilerParams(use_tc_tiling_on_sc=False),
)
def _pool(ids_hbm, table_hbm, out_hbm, ids_v, rows, out_v, sems):
    wid = lax.axis_index("s") * NC + lax.axis_index("c")
    ids_base = wid * (SEQ_PER_W * T)

    def accum(rows_p, s):
        def body(t, acc):
            return tuple(acc[j] + rows_p[t, pl.ds(16 * j, 16)]
                         for j in range(4))
        acc = lax.fori_loop(
            0, T, body,
            tuple(jnp.zeros((16,), jnp.float32) for _ in range(4)),
            unroll=10)
        for j in range(4):
            out_v[s, pl.ds(16 * j, 16)] = acc[j]

    def chunk_body(c, carry):
        del carry
        pltpu.sync_copy(
            ids_hbm.at[pl.ds(pl.multiple_of(ids_base + c * (IDS_CHUNK * T), 8),
                             IDS_CHUNK * T)],
            ids_v)
        seq_base = c * IDS_CHUNK

        for j in range(NBUF - 1):
            _seq_gather(table_hbm, ids_v, rows[j], sems[j], j)

        def step(i, carry):
            del carry
            for p in range(NBUF):
                j = i * NBUF + p
                _seq_wait(table_hbm, ids_v, rows[p], sems[p], j)

                @pl.when(j + NBUF - 1 < IDS_CHUNK)
                def _():
                    _seq_gather(table_hbm, ids_v, rows[(p + NBUF - 1) % NBUF],
                                sems[(p + NBUF - 1) % NBUF], j + NBUF - 1)
                accum(rows[p], seq_base + j)
            return 0

        lax.fori_loop(0, IDS_CHUNK // NBUF, step, 0)
        return 0

    lax.fori_loop(0, N_CHUNKS, chunk_body, 0)
    pltpu.sync_copy(out_v,
                    out_hbm.at[pl.ds(pl.multiple_of(wid * SEQ_PER_W, 8),
                                     SEQ_PER_W)])


_TRB = 2048
_HALF = _TRB // 2
_NBLK = (VOCAB + _TRB - 1) // _TRB
VOCAB2 = _NBLK * _TRB
_IDS_BLK = (B * T + _NBLK - 1) // _NBLK
_IDS_BLK = ((_IDS_BLK + 8191) // 8192) * 8192


def _prep_body(x_ref, ids_ref, o_ref, ids_o_ref):
    xt = x_ref[...].T
    o_ref[...] = jnp.concatenate([xt[:_HALF], xt[_HALF:]], axis=1)
    r = ids_ref[...]
    q = r & (_TRB - 1)
    ids_o_ref[...] = (r - q) + jnp.where(q < _HALF, q + q,
                                         q + q - (_TRB - 1))


def _prep(tT, ids_flat):
    return pl.pallas_call(
        _prep_body,
        grid=(_NBLK,),
        in_specs=[
            pl.BlockSpec((EMB, _TRB), lambda i: (0, i)),
            pl.BlockSpec((_IDS_BLK,), lambda i: (i,)),
        ],
        out_specs=[
            pl.BlockSpec((_HALF, 128), lambda i: (i, 0)),
            pl.BlockSpec((_IDS_BLK,), lambda i: (i,)),
        ],
        out_shape=[
            jax.ShapeDtypeStruct((VOCAB2 // 2, 128), jnp.float32),
            jax.ShapeDtypeStruct((B * T,), jnp.int32),
        ],
    )(tT, ids_flat)


def _mlp_body(x_ref, w1_ref, b1_ref, w2_ref, b2_ref, o_ref):
    x = x_ref[...]
    h = lax.dot_general(x, w1_ref[...], (((1,), (1,)), ((), ())),
                        preferred_element_type=jnp.float32)
    h = jnp.maximum(h * (1.0 / T) + b1_ref[...], 0.0)
    o_ref[...] = lax.dot_general(h, w2_ref[...], (((1,), (1,)), ((), ())),
                                 preferred_element_type=jnp.float32) + b2_ref[...]


_BLK = 2048


def _mlp(sums, W1, b1, W2, b2):
    grid = B // _BLK
    return pl.pallas_call(
        _mlp_body,
        grid=(grid,),
        in_specs=[
            pl.BlockSpec((_BLK, EMB), lambda i: (i, 0)),
            pl.BlockSpec((EMB, EMB), lambda i: (0, 0)),
            pl.BlockSpec((1, EMB), lambda i: (0, 0)),
            pl.BlockSpec((EMB, EMB), lambda i: (0, 0)),
            pl.BlockSpec((1, EMB), lambda i: (0, 0)),
        ],
        out_specs=pl.BlockSpec((_BLK, EMB), lambda i: (i, 0)),
        out_shape=jax.ShapeDtypeStruct((B, EMB), jnp.float32),
    )(sums, W1, b1, W2, b2)


def kernel(token_ids, table, W1, b1, W2, b2):
    t2, ids2 = _prep(table.T, token_ids.reshape(-1))
    sums = _pool(ids2, t2.reshape(VOCAB2, EMB))
    return _mlp(sums, W1, b1.reshape(1, EMB), W2, b2.reshape(1, EMB))
